# baseline (speedup 1.0000x reference)
import functools

import numpy as np
import jax
import jax.numpy as jnp
from jax import lax
from jax.experimental import pallas as pl
from jax.experimental.pallas import tpu as pltpu
from jax.experimental.pallas import tpu_sc as plsc

_BINS = 50
_RATIO_KLD = 0.5
_GROUPS = [(7, 19), (19, 21), (21, 25), (25, 27), (27, 29), (29, 31),
           (31, 34), (34, 38), (38, 50)]

_NC = 2
_NS = 16
_NW = _NC * _NS
_L = 16

_NCOL = 10
_EROWS = 64
_HCOLS = _NCOL * 64
_HWORDS = _L * _HCOLS
_HTOTAL = 2 * _HWORDS


_DENSE_GRID = 8


def _tc_dense_kernel(dd_ref, dt_ref, sc_ref):
    i = pl.program_id(0)
    dd = dd_ref[...]
    dt = dt_ref[...]

    lane50 = lax.broadcasted_iota(jnp.int32, (1, 50), 1)
    diff = dd - dt
    mse_sum = jnp.sum(jnp.where(lane50 < 7, diff * diff, 0.0))

    j50 = lax.broadcasted_iota(jnp.int32, (50, 16), 0)
    g16 = lax.broadcasted_iota(jnp.int32, (50, 16), 1)
    G = jnp.zeros((50, 16), jnp.float32)
    for g, (s, e) in enumerate(_GROUPS):
        G = G + ((g16 == g) & (j50 >= s) & (j50 < e)).astype(jnp.float32)
    E = jnp.exp(dd)
    S = jax.lax.dot_general(E, G, (((1,), (0,)), ((), ())),
                            preferred_element_type=jnp.float32)
    lane16 = lax.broadcasted_iota(jnp.int32, (1, 16), 1)
    lse_sum = jnp.sum(jnp.where(lane16 < 9, jnp.log(S), 0.0))
    tz_sum = jnp.sum(jnp.where(lane50 >= 7, dd * dt, 0.0))

    @pl.when(i == 0)
    def _():
        sc_ref[0] = jnp.float32(0.0)
        sc_ref[1] = jnp.float32(0.0)

    sc_ref[0] = sc_ref[0] + mse_sum
    sc_ref[1] = sc_ref[1] + (lse_sum - tz_sum)


def _sc_hist_body(de, lt, out, xbuf, mmbuf, mbuf, mm_v, edges_v, hist_v,
                  accbuf, rdbuf, shared, redbuf):
    B = de.shape[0] // _NCOL
    chunk = B // _NW
    vecs = chunk // _L
    mm_chunk = B // _NS
    mm_vecs = (mm_chunk * _NCOL) // (_L * _NCOL)
    cid = lax.axis_index("c")
    sid = lax.axis_index("s")
    wid = sid * _NC + cid
    base = wid * chunk

    pltpu.sync_copy(de.at[pl.ds(sid * mm_chunk * _NCOL, mm_chunk * _NCOL)],
                    mmbuf)
    pltpu.sync_copy(de.at[pl.ds(base * _NCOL, chunk * _NCOL)], xbuf)
    pltpu.sync_copy(lt.at[pl.ds(base * 4, chunk * 4)], mbuf)

    lane = lax.iota(jnp.int32, _L)
    x_lane = lane * _NCOL

    for i in range(_NCOL):
        g_base = x_lane + i

        def _mm(v, carry, g_base=g_base):
            acc_lo, acc_hi = carry
            x = plsc.load_gather(mmbuf, [g_base + v * (_L * _NCOL)])
            return (jnp.minimum(acc_lo, x), jnp.maximum(acc_hi, x))

        first = plsc.load_gather(mmbuf, [g_base])
        acc_lo, acc_hi = lax.fori_loop(1, mm_vecs, _mm, (first, first))
        accbuf[pl.ds(i * 2 * _L, _L)] = acc_lo
        accbuf[pl.ds(i * 2 * _L + _L, _L)] = acc_hi

    pltpu.sync_copy(accbuf, shared.at[pl.ds(sid * 512, 512)])
    plsc.subcore_barrier()
    pltpu.sync_copy(shared, rdbuf)

    lo_row = jnp.full((_L,), np.float32(0), jnp.float32)
    hi_row = jnp.full((_L,), np.float32(1), jnp.float32)
    for i in range(_NCOL):
        lo16 = rdbuf[pl.ds(i * 2 * _L, _L)]
        hi16 = rdbuf[pl.ds(i * 2 * _L + _L, _L)]
        for t in range(1, _NS):
            lo16 = jnp.minimum(lo16, rdbuf[pl.ds(t * 512 + i * 2 * _L, _L)])
            hi16 = jnp.maximum(hi16, rdbuf[pl.ds(t * 512 + i * 2 * _L + _L, _L)])
        s_lo = jnp.min(lo16)
        s_hi = jnp.max(hi16)
        flat = s_hi == s_lo
        s_lo = jnp.where(flat, s_lo - 0.5, s_lo)
        s_hi = jnp.where(flat, s_hi + 0.5, s_hi)
        sel = lane == i
        lo_row = jnp.where(sel, jnp.full((_L,), s_lo, jnp.float32), lo_row)
        hi_row = jnp.where(sel, jnp.full((_L,), s_hi, jnp.float32), hi_row)

    zeros16 = jnp.zeros((_L,), jnp.float32)
    mm_v[pl.ds(0, _L)] = zeros16
    mm_v[pl.ds(_L, _L)] = lo_row
    mm_v[pl.ds(2 * _L, _L)] = hi_row

    def _zero(k, carry):
        for u in range(4):
            hist_v[pl.ds(k * (4 * _L) + u * _L, _L)] = zeros16
        return carry

    lax.fori_loop(0, _HTOTAL // (4 * _L), _zero, 0)

    for j in range(_BINS):
        s32 = np.float32(j) / np.float32(_BINS)
        oms32 = np.float32(1) - s32
        e = lo_row * float(oms32) + hi_row * float(s32)
        edges_v[pl.ds((j + 1) * _L, _L)] = e
    edges_v[pl.ds((_BINS + 1) * _L, _L)] = hi_row

    lane_off = lane * _HCOLS
    m_lane = lane * 4 + 1
    full50 = jnp.full((_L,), np.float32(_BINS), jnp.float32)

    for i in range(_NCOL):
        col = jnp.full((_L,), i, jnp.int32)
        loV = plsc.load_gather(mm_v, [col + _L])
        hiV = plsc.load_gather(mm_v, [col + 2 * _L])
        invV = full50 / (hiV - loV)
        colbase = i * 64
        x_base = x_lane + i

        def _body(v, carry, loV=loV, invV=invV,
                  colbase=colbase, x_base=x_base, col=col):
            x = plsc.load_gather(xbuf, [x_base + v * (_L * _NCOL)])
            m = plsc.load_gather(mbuf, [m_lane + v * (_L * 4)])
            t = (x - loV) * invV
            b0 = jnp.minimum(t.astype(jnp.int32), _BINS - 1)
            e_lo = plsc.load_gather(edges_v, [(b0 + 1) * _L + col])
            e_hi = plsc.load_gather(edges_v, [(b0 + 2) * _L + col])
            up = (x >= e_hi).astype(jnp.int32)
            dn = (x < e_lo).astype(jnp.int32)
            b = jnp.minimum(b0 + up - dn, _BINS - 1)
            addr = lane_off + (b + colbase)
            plsc.addupdate_scatter(hist_v, [addr], 1.0 - m)
            plsc.addupdate_scatter(hist_v, [addr + _HWORDS], m)
            return carry

        lax.fori_loop(0, vecs, _body, 0)

    def _lred2(c, carry):
        acc0 = hist_v[pl.ds(c * _L, _L)]
        acc1 = hist_v[pl.ds(_HWORDS + c * _L, _L)]
        for r in range(1, _L):
            acc0 = acc0 + hist_v[pl.ds(r * _HCOLS + c * _L, _L)]
            acc1 = acc1 + hist_v[pl.ds(_HWORDS + r * _HCOLS + c * _L, _L)]
        redbuf[pl.ds(c * _L, _L)] = acc0
        redbuf[pl.ds(_HCOLS + c * _L, _L)] = acc1
        return carry

    lax.fori_loop(0, _HCOLS // _L, _lred2, 0)
    pltpu.sync_copy(redbuf, out.at[wid])


def _make_sc_hist(B):
    mesh = plsc.VectorSubcoreMesh(core_axis_name="c", subcore_axis_name="s",
                                  num_cores=_NC)
    chunk = B // _NW
    return pl.kernel(
        _sc_hist_body,
        out_type=jax.ShapeDtypeStruct((_NW, 2 * _HCOLS), jnp.float32),
        mesh=mesh,
        compiler_params=pltpu.CompilerParams(needs_layout_passes=False),
        scratch_types=[
            pltpu.VMEM((_NCOL * chunk,), jnp.float32),
            pltpu.VMEM(((B // _NS) * _NCOL,), jnp.float32),
            pltpu.VMEM((4 * chunk,), jnp.float32),
            pltpu.VMEM((3 * _L,), jnp.float32),
            pltpu.VMEM((_EROWS * _L,), jnp.float32),
            pltpu.VMEM((_HTOTAL,), jnp.float32),
            pltpu.VMEM((512,), jnp.float32),
            pltpu.VMEM((_NS * 512,), jnp.float32),
            pltpu.VMEM_SHARED((_NS * 512,), jnp.float32),
            pltpu.VMEM((2 * _HCOLS,), jnp.float32),
        ],
    )


def _tc_post_kernel(hist_ref, sc_ref, o_multi, o_mse, o_ce, o_kld):
    H = hist_ref[...]
    nrows = H.shape[0]
    ridx = lax.broadcasted_iota(jnp.int32, (nrows, 1), 0)
    is_single = (ridx % 2) == 0
    wS = jnp.where(is_single, 1.0, 0.0)
    Sc = jnp.sum(H * wS, axis=0)
    Mc = jnp.sum(H * (1.0 - wS), axis=0)

    s_rows = [Sc[i * 64:i * 64 + _BINS] for i in range(_NCOL)]
    m_rows = [Mc[i * 64:i * 64 + _BINS] for i in range(_NCOL)]
    counts_s = jnp.stack(s_rows)
    counts_m = jnp.stack(m_rows)
    n_s = jnp.sum(counts_s[0])
    n_m = jnp.sum(counts_m[0])
    p = counts_s / n_s
    q = counts_m / n_m
    kld = jnp.sum(jnp.where(p > 0, p * jnp.log(p / (q + 1e-10)), 0.0))

    total = n_s + n_m
    numerical_loss = sc_ref[0] / (total * 7.0)
    mse_loss = numerical_loss * 7.0
    ce_loss = sc_ref[1] / total
    alpha = jnp.float32(_RATIO_KLD)
    multi = (1.0 - alpha) * (mse_loss + ce_loss) + alpha * kld
    o_multi[0] = multi
    o_mse[0] = mse_loss
    o_ce[0] = ce_loss
    o_kld[0] = alpha * kld


def kernel(data_encoded, data_decoded, data_true, label_true, batch_size):
    del batch_size
    B = data_encoded.shape[0]

    hist = _make_sc_hist(B)(data_encoded.reshape(-1), label_true.reshape(-1))

    Bc = B // _DENSE_GRID
    scalars = pl.pallas_call(
        _tc_dense_kernel,
        grid=(_DENSE_GRID,),
        out_shape=jax.ShapeDtypeStruct((2,), jnp.float32),
        in_specs=[
            pl.BlockSpec((Bc, 50), lambda i: (i, 0),
                         memory_space=pltpu.VMEM),
            pl.BlockSpec((Bc, 50), lambda i: (i, 0),
                         memory_space=pltpu.VMEM),
        ],
        out_specs=pl.BlockSpec(memory_space=pltpu.SMEM),
    )(data_decoded, data_true)

    out = pl.pallas_call(
        _tc_post_kernel,
        out_shape=tuple(jax.ShapeDtypeStruct((1,), jnp.float32)
                        for _ in range(4)),
        in_specs=[
            pl.BlockSpec(memory_space=pltpu.VMEM),
            pl.BlockSpec(memory_space=pltpu.SMEM),
        ],
        out_specs=tuple(pl.BlockSpec(memory_space=pltpu.SMEM)
                        for _ in range(4)),
    )(hist.reshape(_NW * 2, _HCOLS), scalars)
    return tuple(o.reshape(()) for o in out)


# speedup vs baseline: 1.0265x; 1.0265x over previous
import numpy as np
import jax
import jax.numpy as jnp
from jax import lax
from jax.experimental import pallas as pl
from jax.experimental.pallas import tpu as pltpu
from jax.experimental.pallas import tpu_sc as plsc

_BINS = 50
_RATIO_KLD = 0.5
_GROUPS = [(7, 19), (19, 21), (21, 25), (25, 27), (27, 29), (29, 31),
           (31, 34), (34, 38), (38, 50)]

_NC = 2
_NS = 16
_NW = _NC * _NS
_L = 16

_NCOL = 10
_EROWS = 64
_HCOLS = _NCOL * 64
_HWORDS = _L * _HCOLS
_HTOTAL = 2 * _HWORDS


_DENSE_GRID = 8


def _tc_dense_kernel(dd_ref, dt_ref, sc_ref):
    i = pl.program_id(0)
    dd = dd_ref[...]
    dt = dt_ref[...]

    lane50 = lax.broadcasted_iota(jnp.int32, (1, 50), 1)
    diff = dd - dt
    mse_sum = jnp.sum(jnp.where(lane50 < 7, diff * diff, 0.0))

    j50 = lax.broadcasted_iota(jnp.int32, (50, 16), 0)
    g16 = lax.broadcasted_iota(jnp.int32, (50, 16), 1)
    G = jnp.zeros((50, 16), jnp.float32)
    for g, (s, e) in enumerate(_GROUPS):
        G = G + ((g16 == g) & (j50 >= s) & (j50 < e)).astype(jnp.float32)
    E = jnp.exp(dd)
    S = jax.lax.dot_general(E, G, (((1,), (0,)), ((), ())),
                            preferred_element_type=jnp.float32)
    lane16 = lax.broadcasted_iota(jnp.int32, (1, 16), 1)
    lse_sum = jnp.sum(jnp.where(lane16 < 9, jnp.log(S), 0.0))
    tz_sum = jnp.sum(jnp.where(lane50 >= 7, dd * dt, 0.0))

    @pl.when(i == 0)
    def _():
        sc_ref[0] = jnp.float32(0.0)
        sc_ref[1] = jnp.float32(0.0)

    sc_ref[0] = sc_ref[0] + mse_sum
    sc_ref[1] = sc_ref[1] + (lse_sum - tz_sum)


def _sc_hist_body(de, lt, out, xbuf, mmbuf, mbuf, mm_v, edges_v, hist_v,
                  accbuf, rdbuf, shared, redbuf):
    B = de.shape[0] // _NCOL
    chunk = B // _NW
    vecs = chunk // _L
    mm_chunk = B // _NS
    mm_vecs = (mm_chunk * _NCOL) // (_L * _NCOL)
    cid = lax.axis_index("c")
    sid = lax.axis_index("s")
    wid = sid * _NC + cid
    base = wid * chunk

    pltpu.sync_copy(de.at[pl.ds(sid * mm_chunk * _NCOL, mm_chunk * _NCOL)],
                    mmbuf)
    pltpu.sync_copy(de.at[pl.ds(base * _NCOL, chunk * _NCOL)], xbuf)
    pltpu.sync_copy(lt.at[pl.ds(base * 4, chunk * 4)], mbuf)

    lane = lax.iota(jnp.int32, _L)
    x_lane = lane * _NCOL

    pos_inf = jnp.full((_L,), np.float32(np.inf), jnp.float32)
    neg_inf = jnp.full((_L,), np.float32(-np.inf), jnp.float32)
    for i in range(_NCOL):
        g_base = x_lane + i

        def _mm(v, carry, g_base=g_base):
            acc_lo, acc_hi = carry
            for u in range(4):
                x = plsc.load_gather(
                    mmbuf, [g_base + (v * 4 + u) * (_L * _NCOL)])
                acc_lo = jnp.minimum(acc_lo, x)
                acc_hi = jnp.maximum(acc_hi, x)
            return (acc_lo, acc_hi)

        acc_lo, acc_hi = lax.fori_loop(0, mm_vecs // 4, _mm,
                                       (pos_inf, neg_inf))
        accbuf[pl.ds(i * 2 * _L, _L)] = acc_lo
        accbuf[pl.ds(i * 2 * _L + _L, _L)] = acc_hi

    pltpu.sync_copy(accbuf, shared.at[pl.ds(sid * 512, 512)])
    plsc.subcore_barrier()
    pltpu.sync_copy(shared, rdbuf)

    lo_row = jnp.full((_L,), np.float32(0), jnp.float32)
    hi_row = jnp.full((_L,), np.float32(1), jnp.float32)
    for i in range(_NCOL):
        lo16 = rdbuf[pl.ds(i * 2 * _L, _L)]
        hi16 = rdbuf[pl.ds(i * 2 * _L + _L, _L)]
        for t in range(1, _NS):
            lo16 = jnp.minimum(lo16, rdbuf[pl.ds(t * 512 + i * 2 * _L, _L)])
            hi16 = jnp.maximum(hi16, rdbuf[pl.ds(t * 512 + i * 2 * _L + _L, _L)])
        s_lo = jnp.min(lo16)
        s_hi = jnp.max(hi16)
        flat = s_hi == s_lo
        s_lo = jnp.where(flat, s_lo - 0.5, s_lo)
        s_hi = jnp.where(flat, s_hi + 0.5, s_hi)
        sel = lane == i
        lo_row = jnp.where(sel, jnp.full((_L,), s_lo, jnp.float32), lo_row)
        hi_row = jnp.where(sel, jnp.full((_L,), s_hi, jnp.float32), hi_row)

    zeros16 = jnp.zeros((_L,), jnp.float32)
    mm_v[pl.ds(0, _L)] = zeros16
    mm_v[pl.ds(_L, _L)] = lo_row
    mm_v[pl.ds(2 * _L, _L)] = hi_row

    def _zero(k, carry):
        for u in range(4):
            hist_v[pl.ds(k * (4 * _L) + u * _L, _L)] = zeros16
        return carry

    lax.fori_loop(0, _HTOTAL // (4 * _L), _zero, 0)

    for j in range(_BINS):
        s32 = np.float32(j) / np.float32(_BINS)
        oms32 = np.float32(1) - s32
        e = lo_row * float(oms32) + hi_row * float(s32)
        edges_v[pl.ds((j + 1) * _L, _L)] = e
    edges_v[pl.ds((_BINS + 1) * _L, _L)] = hi_row

    lane_off = lane * _HCOLS
    m_lane = lane * 4 + 1
    full50 = jnp.full((_L,), np.float32(_BINS), jnp.float32)

    for i in range(_NCOL):
        col = jnp.full((_L,), i, jnp.int32)
        loV = plsc.load_gather(mm_v, [col + _L])
        hiV = plsc.load_gather(mm_v, [col + 2 * _L])
        invV = full50 / (hiV - loV)
        colbase = i * 64
        x_base = x_lane + i

        def _body(v, carry, loV=loV, invV=invV,
                  colbase=colbase, x_base=x_base, col=col):
            x = plsc.load_gather(xbuf, [x_base + v * (_L * _NCOL)])
            m = plsc.load_gather(mbuf, [m_lane + v * (_L * 4)])
            t = (x - loV) * invV
            b0 = jnp.minimum(t.astype(jnp.int32), _BINS - 1)
            e_lo = plsc.load_gather(edges_v, [(b0 + 1) * _L + col])
            e_hi = plsc.load_gather(edges_v, [(b0 + 2) * _L + col])
            up = (x >= e_hi).astype(jnp.int32)
            dn = (x < e_lo).astype(jnp.int32)
            b = jnp.minimum(b0 + up - dn, _BINS - 1)
            addr = lane_off + (b + colbase)
            plsc.addupdate_scatter(hist_v, [addr], 1.0 - m)
            plsc.addupdate_scatter(hist_v, [addr + _HWORDS], m)
            return carry

        lax.fori_loop(0, vecs, _body, 0)

    def _lred2(c, carry):
        acc0 = hist_v[pl.ds(c * _L, _L)]
        acc1 = hist_v[pl.ds(_HWORDS + c * _L, _L)]
        for r in range(1, _L):
            acc0 = acc0 + hist_v[pl.ds(r * _HCOLS + c * _L, _L)]
            acc1 = acc1 + hist_v[pl.ds(_HWORDS + r * _HCOLS + c * _L, _L)]
        redbuf[pl.ds(c * _L, _L)] = acc0
        redbuf[pl.ds(_HCOLS + c * _L, _L)] = acc1
        return carry

    lax.fori_loop(0, _HCOLS // _L, _lred2, 0)
    pltpu.sync_copy(redbuf, out.at[wid])


def _make_sc_hist(B):
    mesh = plsc.VectorSubcoreMesh(core_axis_name="c", subcore_axis_name="s",
                                  num_cores=_NC)
    chunk = B // _NW
    return pl.kernel(
        _sc_hist_body,
        out_type=jax.ShapeDtypeStruct((_NW, 2 * _HCOLS), jnp.float32),
        mesh=mesh,
        compiler_params=pltpu.CompilerParams(needs_layout_passes=False),
        scratch_types=[
            pltpu.VMEM((_NCOL * chunk,), jnp.float32),
            pltpu.VMEM(((B // _NS) * _NCOL,), jnp.float32),
            pltpu.VMEM((4 * chunk,), jnp.float32),
            pltpu.VMEM((3 * _L,), jnp.float32),
            pltpu.VMEM((_EROWS * _L,), jnp.float32),
            pltpu.VMEM((_HTOTAL,), jnp.float32),
            pltpu.VMEM((512,), jnp.float32),
            pltpu.VMEM((_NS * 512,), jnp.float32),
            pltpu.VMEM_SHARED((_NS * 512,), jnp.float32),
            pltpu.VMEM((2 * _HCOLS,), jnp.float32),
        ],
    )


def _tc_post_kernel(hist_ref, sc_ref, o_multi, o_mse, o_ce, o_kld):
    H = hist_ref[...]
    nrows = H.shape[0]
    ridx = lax.broadcasted_iota(jnp.int32, (nrows, 1), 0)
    is_single = (ridx % 2) == 0
    wS = jnp.where(is_single, 1.0, 0.0)
    Sc = jnp.sum(H * wS, axis=0)
    Mc = jnp.sum(H * (1.0 - wS), axis=0)

    s_rows = [Sc[i * 64:i * 64 + _BINS] for i in range(_NCOL)]
    m_rows = [Mc[i * 64:i * 64 + _BINS] for i in range(_NCOL)]
    counts_s = jnp.stack(s_rows)
    counts_m = jnp.stack(m_rows)
    n_s = jnp.sum(counts_s[0])
    n_m = jnp.sum(counts_m[0])
    p = counts_s / n_s
    q = counts_m / n_m
    kld = jnp.sum(jnp.where(p > 0, p * jnp.log(p / (q + 1e-10)), 0.0))

    total = n_s + n_m
    numerical_loss = sc_ref[0] / (total * 7.0)
    mse_loss = numerical_loss * 7.0
    ce_loss = sc_ref[1] / total
    alpha = jnp.float32(_RATIO_KLD)
    multi = (1.0 - alpha) * (mse_loss + ce_loss) + alpha * kld
    o_multi[0] = multi
    o_mse[0] = mse_loss
    o_ce[0] = ce_loss
    o_kld[0] = alpha * kld


def kernel(data_encoded, data_decoded, data_true, label_true, batch_size):
    del batch_size
    B = data_encoded.shape[0]

    hist = _make_sc_hist(B)(data_encoded.reshape(-1), label_true.reshape(-1))

    Bc = B // _DENSE_GRID
    scalars = pl.pallas_call(
        _tc_dense_kernel,
        grid=(_DENSE_GRID,),
        out_shape=jax.ShapeDtypeStruct((2,), jnp.float32),
        in_specs=[
            pl.BlockSpec((Bc, 50), lambda i: (i, 0),
                         memory_space=pltpu.VMEM),
            pl.BlockSpec((Bc, 50), lambda i: (i, 0),
                         memory_space=pltpu.VMEM),
        ],
        out_specs=pl.BlockSpec(memory_space=pltpu.SMEM),
    )(data_decoded, data_true)

    out = pl.pallas_call(
        _tc_post_kernel,
        out_shape=tuple(jax.ShapeDtypeStruct((1,), jnp.float32)
                        for _ in range(4)),
        in_specs=[
            pl.BlockSpec(memory_space=pltpu.VMEM),
            pl.BlockSpec(memory_space=pltpu.SMEM),
        ],
        out_specs=tuple(pl.BlockSpec(memory_space=pltpu.SMEM)
                        for _ in range(4)),
    )(hist.reshape(_NW * 2, _HCOLS), scalars)
    return tuple(o.reshape(()) for o in out)


# speedup vs baseline: 1.0291x; 1.0025x over previous
import numpy as np
import jax
import jax.numpy as jnp
from jax import lax
from jax.experimental import pallas as pl
from jax.experimental.pallas import tpu as pltpu
from jax.experimental.pallas import tpu_sc as plsc

_BINS = 50
_RATIO_KLD = 0.5
_GROUPS = [(7, 19), (19, 21), (21, 25), (25, 27), (27, 29), (29, 31),
           (31, 34), (34, 38), (38, 50)]

_NC = 2
_NS = 16
_NW = _NC * _NS
_L = 16

_NCOL = 10
_EROWS = 64
_HCOLS = _NCOL * 64
_HWORDS = _L * _HCOLS
_HTOTAL = 2 * _HWORDS


_DENSE_GRID = 8


def _tc_dense_kernel(dd_ref, dt_ref, sc_ref):
    i = pl.program_id(0)
    dd = dd_ref[...]
    dt = dt_ref[...]

    lane50 = lax.broadcasted_iota(jnp.int32, (1, 50), 1)
    diff = dd - dt
    mse_sum = jnp.sum(jnp.where(lane50 < 7, diff * diff, 0.0))

    j50 = lax.broadcasted_iota(jnp.int32, (50, 16), 0)
    g16 = lax.broadcasted_iota(jnp.int32, (50, 16), 1)
    G = jnp.zeros((50, 16), jnp.float32)
    for g, (s, e) in enumerate(_GROUPS):
        G = G + ((g16 == g) & (j50 >= s) & (j50 < e)).astype(jnp.float32)
    E = jnp.exp(dd)
    S = jax.lax.dot_general(E, G, (((1,), (0,)), ((), ())),
                            preferred_element_type=jnp.float32)
    lane16 = lax.broadcasted_iota(jnp.int32, (1, 16), 1)
    lse_sum = jnp.sum(jnp.where(lane16 < 9, jnp.log(S), 0.0))
    tz_sum = jnp.sum(jnp.where(lane50 >= 7, dd * dt, 0.0))

    @pl.when(i == 0)
    def _():
        sc_ref[0] = jnp.float32(0.0)
        sc_ref[1] = jnp.float32(0.0)

    sc_ref[0] = sc_ref[0] + mse_sum
    sc_ref[1] = sc_ref[1] + (lse_sum - tz_sum)


def _sc_hist_body(de, lt, out, xbuf, mmbuf, mbuf, mm_v, edges_v, hist_v,
                  accbuf, rdbuf, shared, redbuf):
    B = de.shape[0] // _NCOL
    chunk = B // _NW
    vecs = chunk // _L
    mm_chunk = B // _NS
    mm_vecs = (mm_chunk * _NCOL) // (_L * _NCOL)
    cid = lax.axis_index("c")
    sid = lax.axis_index("s")
    wid = sid * _NC + cid
    base = wid * chunk

    pltpu.sync_copy(de.at[pl.ds(sid * mm_chunk * _NCOL, mm_chunk * _NCOL)],
                    mmbuf)
    pltpu.sync_copy(de.at[pl.ds(base * _NCOL, chunk * _NCOL)], xbuf)
    pltpu.sync_copy(lt.at[pl.ds(base * 4, chunk * 4)], mbuf)

    lane = lax.iota(jnp.int32, _L)
    x_lane = lane * _NCOL

    pos_inf = jnp.full((_L,), np.float32(np.inf), jnp.float32)
    neg_inf = jnp.full((_L,), np.float32(-np.inf), jnp.float32)
    for i in range(_NCOL):
        g_base = x_lane + i

        def _mm(v, carry, g_base=g_base):
            acc_lo, acc_hi = carry
            for u in range(4):
                x = plsc.load_gather(
                    mmbuf, [g_base + (v * 4 + u) * (_L * _NCOL)])
                acc_lo = jnp.minimum(acc_lo, x)
                acc_hi = jnp.maximum(acc_hi, x)
            return (acc_lo, acc_hi)

        acc_lo, acc_hi = lax.fori_loop(0, mm_vecs // 4, _mm,
                                       (pos_inf, neg_inf))
        accbuf[pl.ds(i * 2 * _L, _L)] = acc_lo
        accbuf[pl.ds(i * 2 * _L + _L, _L)] = acc_hi

    pltpu.sync_copy(accbuf, shared.at[pl.ds(sid * 512, 512)])
    plsc.subcore_barrier()
    pltpu.sync_copy(shared, rdbuf)

    lo_row = jnp.full((_L,), np.float32(0), jnp.float32)
    hi_row = jnp.full((_L,), np.float32(1), jnp.float32)
    for i in range(_NCOL):
        lo16 = rdbuf[pl.ds(i * 2 * _L, _L)]
        hi16 = rdbuf[pl.ds(i * 2 * _L + _L, _L)]
        for t in range(1, _NS):
            lo16 = jnp.minimum(lo16, rdbuf[pl.ds(t * 512 + i * 2 * _L, _L)])
            hi16 = jnp.maximum(hi16, rdbuf[pl.ds(t * 512 + i * 2 * _L + _L, _L)])
        s_lo = jnp.min(lo16)
        s_hi = jnp.max(hi16)
        flat = s_hi == s_lo
        s_lo = jnp.where(flat, s_lo - 0.5, s_lo)
        s_hi = jnp.where(flat, s_hi + 0.5, s_hi)
        sel = lane == i
        lo_row = jnp.where(sel, jnp.full((_L,), s_lo, jnp.float32), lo_row)
        hi_row = jnp.where(sel, jnp.full((_L,), s_hi, jnp.float32), hi_row)

    zeros16 = jnp.zeros((_L,), jnp.float32)
    mm_v[pl.ds(0, _L)] = zeros16
    mm_v[pl.ds(_L, _L)] = lo_row
    mm_v[pl.ds(2 * _L, _L)] = hi_row

    def _zero(k, carry):
        for u in range(4):
            hist_v[pl.ds(k * (4 * _L) + u * _L, _L)] = zeros16
        return carry

    lax.fori_loop(0, _HTOTAL // (4 * _L), _zero, 0)

    for j in range(_BINS):
        s32 = np.float32(j) / np.float32(_BINS)
        oms32 = np.float32(1) - s32
        e = lo_row * float(oms32) + hi_row * float(s32)
        edges_v[pl.ds((j + 1) * _L, _L)] = e
    edges_v[pl.ds((_BINS + 1) * _L, _L)] = hi_row

    lane_off = lane * _HCOLS
    m_lane = lane * 4 + 1
    full50 = jnp.full((_L,), np.float32(_BINS), jnp.float32)

    for i in range(_NCOL):
        col = jnp.full((_L,), i, jnp.int32)
        loV = plsc.load_gather(mm_v, [col + _L])
        hiV = plsc.load_gather(mm_v, [col + 2 * _L])
        invV = full50 / (hiV - loV)
        colbase = i * 64
        x_base = x_lane + i

        def _body(v, carry, loV=loV, invV=invV,
                  colbase=colbase, x_base=x_base, col=col):
            for u in range(2):
                w = v * 2 + u
                x = plsc.load_gather(xbuf, [x_base + w * (_L * _NCOL)])
                m = plsc.load_gather(mbuf, [m_lane + w * (_L * 4)])
                t = (x - loV) * invV
                b0 = jnp.minimum(t.astype(jnp.int32), _BINS - 1)
                e_lo = plsc.load_gather(edges_v, [(b0 + 1) * _L + col])
                e_hi = plsc.load_gather(edges_v, [(b0 + 2) * _L + col])
                up = (x >= e_hi).astype(jnp.int32)
                dn = (x < e_lo).astype(jnp.int32)
                b = jnp.minimum(b0 + up - dn, _BINS - 1)
                addr = lane_off + (b + colbase)
                plsc.addupdate_scatter(hist_v, [addr], 1.0 - m)
                plsc.addupdate_scatter(hist_v, [addr + _HWORDS], m)
            return carry

        lax.fori_loop(0, vecs // 2, _body, 0)

    def _lred2(c, carry):
        acc0 = hist_v[pl.ds(c * _L, _L)]
        acc1 = hist_v[pl.ds(_HWORDS + c * _L, _L)]
        for r in range(1, _L):
            acc0 = acc0 + hist_v[pl.ds(r * _HCOLS + c * _L, _L)]
            acc1 = acc1 + hist_v[pl.ds(_HWORDS + r * _HCOLS + c * _L, _L)]
        redbuf[pl.ds(c * _L, _L)] = acc0
        redbuf[pl.ds(_HCOLS + c * _L, _L)] = acc1
        return carry

    lax.fori_loop(0, _HCOLS // _L, _lred2, 0)
    pltpu.sync_copy(redbuf, out.at[wid])


def _make_sc_hist(B):
    mesh = plsc.VectorSubcoreMesh(core_axis_name="c", subcore_axis_name="s",
                                  num_cores=_NC)
    chunk = B // _NW
    return pl.kernel(
        _sc_hist_body,
        out_type=jax.ShapeDtypeStruct((_NW, 2 * _HCOLS), jnp.float32),
        mesh=mesh,
        compiler_params=pltpu.CompilerParams(needs_layout_passes=False),
        scratch_types=[
            pltpu.VMEM((_NCOL * chunk,), jnp.float32),
            pltpu.VMEM(((B // _NS) * _NCOL,), jnp.float32),
            pltpu.VMEM((4 * chunk,), jnp.float32),
            pltpu.VMEM((3 * _L,), jnp.float32),
            pltpu.VMEM((_EROWS * _L,), jnp.float32),
            pltpu.VMEM((_HTOTAL,), jnp.float32),
            pltpu.VMEM((512,), jnp.float32),
            pltpu.VMEM((_NS * 512,), jnp.float32),
            pltpu.VMEM_SHARED((_NS * 512,), jnp.float32),
            pltpu.VMEM((2 * _HCOLS,), jnp.float32),
        ],
    )


def _tc_post_kernel(hist_ref, sc_ref, o_multi, o_mse, o_ce, o_kld):
    H = hist_ref[...]
    nrows = H.shape[0]
    ridx = lax.broadcasted_iota(jnp.int32, (nrows, 1), 0)
    is_single = (ridx % 2) == 0
    wS = jnp.where(is_single, 1.0, 0.0)
    Sc = jnp.sum(H * wS, axis=0)
    Mc = jnp.sum(H * (1.0 - wS), axis=0)

    s_rows = [Sc[i * 64:i * 64 + _BINS] for i in range(_NCOL)]
    m_rows = [Mc[i * 64:i * 64 + _BINS] for i in range(_NCOL)]
    counts_s = jnp.stack(s_rows)
    counts_m = jnp.stack(m_rows)
    n_s = jnp.sum(counts_s[0])
    n_m = jnp.sum(counts_m[0])
    p = counts_s / n_s
    q = counts_m / n_m
    kld = jnp.sum(jnp.where(p > 0, p * jnp.log(p / (q + 1e-10)), 0.0))

    total = n_s + n_m
    numerical_loss = sc_ref[0] / (total * 7.0)
    mse_loss = numerical_loss * 7.0
    ce_loss = sc_ref[1] / total
    alpha = jnp.float32(_RATIO_KLD)
    multi = (1.0 - alpha) * (mse_loss + ce_loss) + alpha * kld
    o_multi[0] = multi
    o_mse[0] = mse_loss
    o_ce[0] = ce_loss
    o_kld[0] = alpha * kld


def kernel(data_encoded, data_decoded, data_true, label_true, batch_size):
    del batch_size
    B = data_encoded.shape[0]

    hist = _make_sc_hist(B)(data_encoded.reshape(-1), label_true.reshape(-1))

    Bc = B // _DENSE_GRID
    scalars = pl.pallas_call(
        _tc_dense_kernel,
        grid=(_DENSE_GRID,),
        out_shape=jax.ShapeDtypeStruct((2,), jnp.float32),
        in_specs=[
            pl.BlockSpec((Bc, 50), lambda i: (i, 0),
                         memory_space=pltpu.VMEM),
            pl.BlockSpec((Bc, 50), lambda i: (i, 0),
                         memory_space=pltpu.VMEM),
        ],
        out_specs=pl.BlockSpec(memory_space=pltpu.SMEM),
    )(data_decoded, data_true)

    out = pl.pallas_call(
        _tc_post_kernel,
        out_shape=tuple(jax.ShapeDtypeStruct((1,), jnp.float32)
                        for _ in range(4)),
        in_specs=[
            pl.BlockSpec(memory_space=pltpu.VMEM),
            pl.BlockSpec(memory_space=pltpu.SMEM),
        ],
        out_specs=tuple(pl.BlockSpec(memory_space=pltpu.SMEM)
                        for _ in range(4)),
    )(hist.reshape(_NW * 2, _HCOLS), scalars)
    return tuple(o.reshape(()) for o in out)
